# baseline (speedup 1.0000x reference)
import functools

import jax
import jax.numpy as jnp
from jax import lax
from jax.experimental import pallas as pl
from jax.experimental.pallas import tpu as pltpu
from jax.experimental.pallas import tpu_sc as plsc

D = 128
L = 16
KD = D // L
C = 128
MAXCH = 7
NSLOT = 7
RECW = 8
NEG_INF = float("-inf")

_GATHER_DNUMS = lax.GatherDimensionNumbers(
    offset_dims=(), collapsed_slice_dims=(0,), start_index_map=(0,))


def _take(v, idx):
    return lax.gather(v, idx[:, None], _GATHER_DNUMS, (1,),
                      mode=lax.GatherScatterMode.PROMISE_IN_BOUNDS)


def _tree_sum(v, lane):
    for sh in (1, 2, 4, 8):
        v = v + _take(v, lane ^ sh)
    return v


def _tree_max(v, lane):
    for sh in (1, 2, 4, 8):
        v = jnp.maximum(v, _take(v, lane ^ sh))
    return v


def _splat_lane(v, lane, j):
    return _take(v, (lane & 0) + j)


def _sc_body(n_rows, trows, rounds, feat_hbm, tbl_hbm, w_hbm, out_hbm,
             buf, my_tbl, w_v, orow, csems):
    cid = lax.axis_index("c")
    sid = lax.axis_index("s")
    wid = sid * 2 + cid

    pltpu.sync_copy(tbl_hbm.at[pl.ds(wid * trows * RECW, trows * RECW)],
                    my_tbl)
    pltpu.sync_copy(w_hbm, w_v)

    lane = lax.iota(jnp.int32, L)
    w_regs = [w_v[pl.ds(kk * L, L)] for kk in range(KD)]
    zero_v = jnp.zeros((L,), jnp.float32)
    ninf_v = jnp.full((L,), NEG_INF, jnp.float32)

    def fire(rec, slot):
        @pl.when(rec[6] > 0)
        def _():
            pltpu.make_async_copy(
                feat_hbm.at[pl.ds(rec[0] * D, C * D)],
                buf.at[pl.ds(slot * C * D, C * D)],
                csems.at[slot],
            ).start()

    for r in range(NSLOT):
        fire(my_tbl[pl.ds(r * RECW, L)], r)

    def do_round(t, carry):
        m, dvec = carry[0], carry[1]
        accs = list(carry[2:])
        for r in range(NSLOT):
            c = t * NSLOT + r
            rec = my_tbl[pl.ds(c * RECW, L)]
            j0, j1, seg = rec[1], rec[2], rec[3]

            @pl.when(rec[6] > 0)
            def _wait(r=r):
                pltpu.make_async_copy(
                    feat_hbm.at[pl.ds(0, C * D)],
                    buf.at[pl.ds(r * C * D, C * D)],
                    csems.at[r],
                ).wait()

            fs = rec[4] > 0
            m = jnp.where(fs, ninf_v, m)
            dvec = jnp.where(fs, zero_v, dvec)
            accs = [jnp.where(fs, zero_v, a) for a in accs]

            def sub_body(sg, carry, r=r):
                m, dvec, j0, j1 = carry[0], carry[1], carry[2], carry[3]
                accs = list(carry[4:])
                for half in range(2):
                    jbase = sg * 8 + half * 4
                    q0 = r * C + jbase
                    xr = [[buf[pl.ds((q0 + jj) * D + kk * L, L)]
                           for kk in range(KD)] for jj in range(4)]
                    svec = ninf_v
                    for jj in range(4):
                        a = w_regs[0] * xr[jj][0]
                        for kk in range(1, KD):
                            a = a + w_regs[kk] * xr[jj][kk]
                        sv = _tree_sum(a, lane)
                        sv = jnp.maximum(sv, 0.2 * sv)
                        svec = jnp.where(lane == jj, sv, svec)
                    jvec = jbase + lane
                    valid = (jvec >= j0) & (jvec < j1)
                    svec = jnp.where(valid, svec, NEG_INF)
                    m_new = jnp.maximum(m, _tree_max(svec, lane))
                    m_new_g = jnp.where(m_new == NEG_INF, 0.0, m_new)
                    m_old_g = jnp.where(m == NEG_INF, 0.0, m)
                    scale = jnp.exp(m_old_g - m_new_g)
                    pvec = jnp.exp(svec - m_new_g)
                    dvec = dvec * scale + pvec
                    pbs = [_splat_lane(pvec, lane, jj) for jj in range(4)]
                    for kk in range(KD):
                        accs[kk] = (accs[kk] * scale + pbs[0] * xr[0][kk]
                                    + pbs[1] * xr[1][kk] + pbs[2] * xr[2][kk]
                                    + pbs[3] * xr[3][kk])
                    m = m_new
                return (m, dvec, j0, j1, *accs)

            out_c = lax.fori_loop(0, jnp.where(rec[7] > 0, C // 8, 0),
                                  sub_body, (m, dvec, j0, j1, *accs))
            m, dvec = out_c[0], out_c[1]
            accs = list(out_c[4:])

            @pl.when(rec[5] > 0)
            def _finish(accs=accs, dvec=dvec, seg=seg):
                inv = 1.0 / jnp.maximum(_tree_sum(dvec, lane), 1e-30)
                for kk in range(KD):
                    orow[pl.ds(kk * L, L)] = accs[kk] * inv
                pltpu.sync_copy(orow, out_hbm.at[pl.ds(seg * D, D)])

            fire(my_tbl[pl.ds((c + NSLOT) * RECW, L)], r)
        return (m, dvec, *accs)

    lax.fori_loop(0, rounds, do_round,
                  (ninf_v, zero_v, *([zero_v] * KD)))


def _chunk_table(sizes, n_rows, nw):
    b = sizes.shape[0]
    kpad = -(-b // nw)
    pad = kpad * nw - b
    offsets = jnp.concatenate(
        [jnp.zeros((1,), jnp.int32), jnp.cumsum(sizes, dtype=jnp.int32)])
    starts = jnp.concatenate([offsets[:b], jnp.zeros((pad,), jnp.int32)])
    szs = jnp.concatenate(
        [sizes.astype(jnp.int32), jnp.full((pad,), -1, jnp.int32)])
    seg_ids = jnp.arange(kpad * nw, dtype=jnp.int32)

    ci_ax = jnp.arange(MAXCH, dtype=jnp.int32).reshape(1, 1, MAXCH)
    g_start = starts.reshape(kpad, nw, 1)
    g_size = szs.reshape(kpad, nw, 1)
    g_id = seg_ids.reshape(kpad, nw, 1)

    nch = -(-g_size // C)
    s_i = g_start + ci_ax * C
    base = jnp.minimum(s_i, n_rows - C)
    j0 = s_i - base
    j1 = jnp.minimum(jnp.minimum(s_i + C, g_start + g_size) - base, C)
    is_dma = (ci_ax < nch) & (g_size > 0)
    present = is_dma | ((ci_ax == 0) & (g_size == 0))
    first = present & (ci_ax == 0)
    last = present & (((ci_ax == nch - 1) & (g_size > 0)) | (g_size == 0))
    rec = jnp.stack([
        jnp.where(is_dma, base, 0),
        jnp.where(is_dma, j0, 0),
        jnp.where(is_dma, j1, 0),
        jnp.broadcast_to(g_id, is_dma.shape),
        first.astype(jnp.int32),
        last.astype(jnp.int32),
        is_dma.astype(jnp.int32),
        present.astype(jnp.int32),
    ], axis=-1)

    rec = jnp.transpose(rec, (1, 0, 2, 3)).reshape(nw, kpad * MAXCH, RECW)
    presentw = jnp.transpose(present, (1, 0, 2)).reshape(nw, kpad * MAXCH)
    rec = rec * presentw[:, :, None].astype(jnp.int32)

    maxc = kpad * MAXCH
    rounds = -(-maxc // NSLOT)
    trows = rounds * NSLOT + NSLOT + 2
    rec = jnp.pad(rec, ((0, 0), (0, trows - maxc), (0, 0)))
    return rec.reshape(-1), rounds, trows


B_SPLIT_FRAC = 0.56


KSEG = 4


def _tc_body(b0, n_rows, win, offs_ref, feat_hbm, w_ref, out_ref, buf, sems):
    t = pl.program_id(0)
    n_steps = pl.num_programs(0)

    def window_base(ts):
        return jnp.minimum(offs_ref[b0 + ts * KSEG], n_rows - win)

    def start_copy(ts, slot):
        pltpu.make_async_copy(
            feat_hbm.at[pl.ds(window_base(ts), win), :],
            buf.at[slot],
            sems.at[slot],
        ).start()

    @pl.when(t == 0)
    def _prologue():
        start_copy(0, 0)

    @pl.when(t + 1 < n_steps)
    def _prefetch_next():
        start_copy(t + 1, (t + 1) % 2)

    slot = t % 2
    pltpu.make_async_copy(
        feat_hbm.at[pl.ds(window_base(t), win), :],
        buf.at[slot],
        sems.at[slot],
    ).wait()

    base = window_base(t)
    x = buf[slot]
    gidx = base + lax.broadcasted_iota(jnp.int32, (1, win), 1)

    s_row = lax.dot_general(w_ref[...], x, (((1,), (1,)), ((), ())),
                            preferred_element_type=jnp.float32)
    s_row = jnp.where(s_row >= 0, s_row, 0.2 * s_row)

    for i in range(KSEG):
        start = offs_ref[b0 + t * KSEG + i]
        end = offs_ref[b0 + t * KSEG + i + 1]
        mask = (gidx >= start) & (gidx < end)
        s = jnp.where(mask, s_row, -jnp.inf)
        mx = jnp.max(s)
        mx = jnp.where(jnp.isfinite(mx), mx, 0.0)
        pr = jnp.where(mask, jnp.exp(s - mx), 0.0)
        denom = jnp.maximum(jnp.sum(pr), 1e-30)
        acc = lax.dot_general(pr, x, (((1,), (0,)), ((), ())),
                              preferred_element_type=jnp.float32)
        out_ref[i] = acc / denom


def _tc_big_segments(feat, offsets, W, b0, b):
    n_rows, d = feat.shape
    win = 800 * KSEG
    n_tc = b - b0
    assert n_tc % KSEG == 0
    grid_spec = pltpu.PrefetchScalarGridSpec(
        num_scalar_prefetch=1,
        grid=(n_tc // KSEG,),
        in_specs=[
            pl.BlockSpec(memory_space=pl.ANY),
            pl.BlockSpec((1, d), lambda g, offs: (0, 0)),
        ],
        out_specs=pl.BlockSpec((KSEG, 1, d), lambda g, offs: (g, 0, 0)),
        scratch_shapes=[
            pltpu.VMEM((2, win, d), jnp.float32),
            pltpu.SemaphoreType.DMA((2,)),
        ],
    )
    out = pl.pallas_call(
        functools.partial(_tc_body, b0, n_rows, win),
        grid_spec=grid_spec,
        out_shape=jax.ShapeDtypeStruct((n_tc, 1, d), jnp.float32),
    )(offsets, feat, W.reshape(1, d))
    return out.reshape(n_tc, d)


def kernel(feat, sizes, W):
    n_rows, d = feat.shape
    b = sizes.shape[0]
    nw = 32
    b0 = (int(b * B_SPLIT_FRAC) // nw) * nw

    offsets = jnp.concatenate(
        [jnp.zeros((1,), jnp.int32), jnp.cumsum(sizes, dtype=jnp.int32)])
    tbl, rounds, trows = _chunk_table(sizes[:b0], n_rows, nw)
    w_flat = W.reshape(d)

    mesh = plsc.VectorSubcoreMesh(core_axis_name="c", subcore_axis_name="s")
    body = functools.partial(_sc_body, n_rows, trows, rounds)
    run = pl.kernel(
        body,
        out_type=jax.ShapeDtypeStruct((b0 * d,), jnp.float32),
        mesh=mesh,
        scratch_types=[
            pltpu.VMEM((NSLOT * C * D,), jnp.float32),
            pltpu.VMEM((trows * RECW,), jnp.int32),
            pltpu.VMEM((d,), jnp.float32),
            pltpu.VMEM((d,), jnp.float32),
            pltpu.SemaphoreType.DMA((NSLOT,)),
        ],
    )
    out_sc = run(feat.reshape(-1), tbl, w_flat)
    out_tc = _tc_big_segments(feat, offsets, W, b0, b)
    return jnp.concatenate([out_sc.reshape(b0, d), out_tc], axis=0)


# speedup vs baseline: 1.0079x; 1.0079x over previous
import functools

import jax
import jax.numpy as jnp
from jax import lax
from jax.experimental import pallas as pl
from jax.experimental.pallas import tpu as pltpu
from jax.experimental.pallas import tpu_sc as plsc

D = 128
L = 16
KD = D // L
C = 128
MAXCH = 7
NSLOT = 7
RECW = 8
NEG_INF = float("-inf")

_GATHER_DNUMS = lax.GatherDimensionNumbers(
    offset_dims=(), collapsed_slice_dims=(0,), start_index_map=(0,))


def _take(v, idx):
    return lax.gather(v, idx[:, None], _GATHER_DNUMS, (1,),
                      mode=lax.GatherScatterMode.PROMISE_IN_BOUNDS)


def _tree_sum(v, lane):
    for sh in (1, 2, 4, 8):
        v = v + _take(v, lane ^ sh)
    return v


def _tree_max(v, lane):
    for sh in (1, 2, 4, 8):
        v = jnp.maximum(v, _take(v, lane ^ sh))
    return v


def _splat_lane(v, lane, j):
    return _take(v, (lane & 0) + j)


def _sc_body(n_rows, trows, rounds, feat_hbm, tbl_hbm, w_hbm, out_hbm,
             buf, my_tbl, w_v, orow, csems):
    cid = lax.axis_index("c")
    sid = lax.axis_index("s")
    wid = sid * 2 + cid

    pltpu.sync_copy(tbl_hbm.at[pl.ds(wid * trows * RECW, trows * RECW)],
                    my_tbl)
    pltpu.sync_copy(w_hbm, w_v)

    lane = lax.iota(jnp.int32, L)
    w_regs = [w_v[pl.ds(kk * L, L)] for kk in range(KD)]
    zero_v = jnp.zeros((L,), jnp.float32)
    ninf_v = jnp.full((L,), NEG_INF, jnp.float32)

    def fire(rec, slot):
        @pl.when(rec[6] > 0)
        def _():
            pltpu.make_async_copy(
                feat_hbm.at[pl.ds(rec[0] * D, C * D)],
                buf.at[pl.ds(slot * C * D, C * D)],
                csems.at[slot],
            ).start()

    for r in range(NSLOT):
        fire(my_tbl[pl.ds(r * RECW, L)], r)

    def do_round(t, carry):
        m, dvec = carry[0], carry[1]
        accs = list(carry[2:])
        for r in range(NSLOT):
            c = t * NSLOT + r
            rec = my_tbl[pl.ds(c * RECW, L)]
            j0, j1, seg = rec[1], rec[2], rec[3]

            @pl.when(rec[6] > 0)
            def _wait(r=r):
                pltpu.make_async_copy(
                    feat_hbm.at[pl.ds(0, C * D)],
                    buf.at[pl.ds(r * C * D, C * D)],
                    csems.at[r],
                ).wait()

            fs = rec[4] > 0
            m = jnp.where(fs, ninf_v, m)
            dvec = jnp.where(fs, zero_v, dvec)
            accs = [jnp.where(fs, zero_v, a) for a in accs]

            def sub_body(sg, carry, r=r):
                m, dvec, j0, j1 = carry[0], carry[1], carry[2], carry[3]
                accs = list(carry[4:])
                for half in range(2):
                    jbase = sg * 8 + half * 4
                    q0 = r * C + jbase
                    xr = [[buf[pl.ds((q0 + jj) * D + kk * L, L)]
                           for kk in range(KD)] for jj in range(4)]
                    svec = ninf_v
                    for jj in range(4):
                        a = w_regs[0] * xr[jj][0]
                        for kk in range(1, KD):
                            a = a + w_regs[kk] * xr[jj][kk]
                        sv = _tree_sum(a, lane)
                        sv = jnp.maximum(sv, 0.2 * sv)
                        svec = jnp.where(lane == jj, sv, svec)
                    jvec = jbase + lane
                    valid = (jvec >= j0) & (jvec < j1)
                    svec = jnp.where(valid, svec, NEG_INF)
                    m_new = jnp.maximum(m, _tree_max(svec, lane))
                    m_new_g = jnp.where(m_new == NEG_INF, 0.0, m_new)
                    m_old_g = jnp.where(m == NEG_INF, 0.0, m)
                    scale = jnp.exp(m_old_g - m_new_g)
                    pvec = jnp.exp(svec - m_new_g)
                    dvec = dvec * scale + pvec
                    pbs = [_splat_lane(pvec, lane, jj) for jj in range(4)]
                    for kk in range(KD):
                        accs[kk] = (accs[kk] * scale + pbs[0] * xr[0][kk]
                                    + pbs[1] * xr[1][kk] + pbs[2] * xr[2][kk]
                                    + pbs[3] * xr[3][kk])
                    m = m_new
                return (m, dvec, j0, j1, *accs)

            out_c = lax.fori_loop(0, jnp.where(rec[7] > 0, C // 8, 0),
                                  sub_body, (m, dvec, j0, j1, *accs))
            m, dvec = out_c[0], out_c[1]
            accs = list(out_c[4:])

            @pl.when(rec[5] > 0)
            def _finish(accs=accs, dvec=dvec, seg=seg):
                inv = 1.0 / jnp.maximum(_tree_sum(dvec, lane), 1e-30)
                for kk in range(KD):
                    orow[pl.ds(kk * L, L)] = accs[kk] * inv
                pltpu.sync_copy(orow, out_hbm.at[pl.ds(seg * D, D)])

            fire(my_tbl[pl.ds((c + NSLOT) * RECW, L)], r)
        return (m, dvec, *accs)

    lax.fori_loop(0, rounds, do_round,
                  (ninf_v, zero_v, *([zero_v] * KD)))


def _chunk_table(sizes, n_rows, nw):
    b = sizes.shape[0]
    kpad = -(-b // nw)
    pad = kpad * nw - b
    offsets = jnp.concatenate(
        [jnp.zeros((1,), jnp.int32), jnp.cumsum(sizes, dtype=jnp.int32)])
    starts = jnp.concatenate([offsets[:b], jnp.zeros((pad,), jnp.int32)])
    szs = jnp.concatenate(
        [sizes.astype(jnp.int32), jnp.full((pad,), -1, jnp.int32)])
    seg_ids = jnp.arange(kpad * nw, dtype=jnp.int32)

    ci_ax = jnp.arange(MAXCH, dtype=jnp.int32).reshape(1, 1, MAXCH)
    g_start = starts.reshape(kpad, nw, 1)
    g_size = szs.reshape(kpad, nw, 1)
    g_id = seg_ids.reshape(kpad, nw, 1)

    nch = -(-g_size // C)
    s_i = g_start + ci_ax * C
    base = jnp.minimum(s_i, n_rows - C)
    j0 = s_i - base
    j1 = jnp.minimum(jnp.minimum(s_i + C, g_start + g_size) - base, C)
    is_dma = (ci_ax < nch) & (g_size > 0)
    present = is_dma | ((ci_ax == 0) & (g_size == 0))
    first = present & (ci_ax == 0)
    last = present & (((ci_ax == nch - 1) & (g_size > 0)) | (g_size == 0))
    rec = jnp.stack([
        jnp.where(is_dma, base, 0),
        jnp.where(is_dma, j0, 0),
        jnp.where(is_dma, j1, 0),
        jnp.broadcast_to(g_id, is_dma.shape),
        first.astype(jnp.int32),
        last.astype(jnp.int32),
        is_dma.astype(jnp.int32),
        present.astype(jnp.int32),
    ], axis=-1)

    rec = jnp.transpose(rec, (1, 0, 2, 3)).reshape(nw, kpad * MAXCH, RECW)
    presentw = jnp.transpose(present, (1, 0, 2)).reshape(nw, kpad * MAXCH)
    rec = rec * presentw[:, :, None].astype(jnp.int32)

    maxc = kpad * MAXCH
    rounds = -(-maxc // NSLOT)
    trows = rounds * NSLOT + NSLOT + 2
    rec = jnp.pad(rec, ((0, 0), (0, trows - maxc), (0, 0)))
    return rec.reshape(-1), rounds, trows


B_SPLIT_FRAC = 0.64


KSEG = 4


def _tc_body(b0, n_rows, win, offs_ref, feat_hbm, w_ref, out_ref, buf, sems):
    t = pl.program_id(0)
    n_steps = pl.num_programs(0)

    def window_base(ts):
        return jnp.minimum(offs_ref[b0 + ts * KSEG], n_rows - win)

    def start_copy(ts, slot):
        pltpu.make_async_copy(
            feat_hbm.at[pl.ds(window_base(ts), win), :],
            buf.at[slot],
            sems.at[slot],
        ).start()

    @pl.when(t == 0)
    def _prologue():
        start_copy(0, 0)

    @pl.when(t + 1 < n_steps)
    def _prefetch_next():
        start_copy(t + 1, (t + 1) % 2)

    slot = t % 2
    pltpu.make_async_copy(
        feat_hbm.at[pl.ds(window_base(t), win), :],
        buf.at[slot],
        sems.at[slot],
    ).wait()

    base = window_base(t)
    x = buf[slot]
    gidx = base + lax.broadcasted_iota(jnp.int32, (1, win), 1)

    s_row = lax.dot_general(w_ref[...], x, (((1,), (1,)), ((), ())),
                            preferred_element_type=jnp.float32)
    s_row = jnp.where(s_row >= 0, s_row, 0.2 * s_row)

    for i in range(KSEG):
        start = offs_ref[b0 + t * KSEG + i]
        end = offs_ref[b0 + t * KSEG + i + 1]
        mask = (gidx >= start) & (gidx < end)
        s = jnp.where(mask, s_row, -jnp.inf)
        mx = jnp.max(s)
        mx = jnp.where(jnp.isfinite(mx), mx, 0.0)
        pr = jnp.where(mask, jnp.exp(s - mx), 0.0)
        denom = jnp.maximum(jnp.sum(pr), 1e-30)
        acc = lax.dot_general(pr, x, (((1,), (0,)), ((), ())),
                              preferred_element_type=jnp.float32)
        out_ref[i] = acc / denom


def _tc_big_segments(feat, offsets, W, b0, b):
    n_rows, d = feat.shape
    win = 800 * KSEG
    n_tc = b - b0
    assert n_tc % KSEG == 0
    grid_spec = pltpu.PrefetchScalarGridSpec(
        num_scalar_prefetch=1,
        grid=(n_tc // KSEG,),
        in_specs=[
            pl.BlockSpec(memory_space=pl.ANY),
            pl.BlockSpec((1, d), lambda g, offs: (0, 0)),
        ],
        out_specs=pl.BlockSpec((KSEG, 1, d), lambda g, offs: (g, 0, 0)),
        scratch_shapes=[
            pltpu.VMEM((2, win, d), jnp.float32),
            pltpu.SemaphoreType.DMA((2,)),
        ],
    )
    out = pl.pallas_call(
        functools.partial(_tc_body, b0, n_rows, win),
        grid_spec=grid_spec,
        out_shape=jax.ShapeDtypeStruct((n_tc, 1, d), jnp.float32),
    )(offsets, feat, W.reshape(1, d))
    return out.reshape(n_tc, d)


def kernel(feat, sizes, W):
    n_rows, d = feat.shape
    b = sizes.shape[0]
    nw = 32
    b0 = (int(b * B_SPLIT_FRAC) // nw) * nw

    offsets = jnp.concatenate(
        [jnp.zeros((1,), jnp.int32), jnp.cumsum(sizes, dtype=jnp.int32)])
    tbl, rounds, trows = _chunk_table(sizes[:b0], n_rows, nw)
    w_flat = W.reshape(d)

    mesh = plsc.VectorSubcoreMesh(core_axis_name="c", subcore_axis_name="s")
    body = functools.partial(_sc_body, n_rows, trows, rounds)
    run = pl.kernel(
        body,
        out_type=jax.ShapeDtypeStruct((b0 * d,), jnp.float32),
        mesh=mesh,
        scratch_types=[
            pltpu.VMEM((NSLOT * C * D,), jnp.float32),
            pltpu.VMEM((trows * RECW,), jnp.int32),
            pltpu.VMEM((d,), jnp.float32),
            pltpu.VMEM((d,), jnp.float32),
            pltpu.SemaphoreType.DMA((NSLOT,)),
        ],
    )
    out_sc = run(feat.reshape(-1), tbl, w_flat)
    out_tc = _tc_big_segments(feat, offsets, W, b0, b)
    return jnp.concatenate([out_sc.reshape(b0, d), out_tc], axis=0)
